# Initial kernel scaffold; baseline (speedup 1.0000x reference)
import jax
import jax.numpy as jnp
from jax.experimental import pallas as pl


def kernel(lv, neighbor_idx, gamma, beta, W, b):
    raise NotImplementedError("write your pallas kernel here")


# speedup vs baseline: 2.1697x; 2.1697x over previous
import functools

import jax
import jax.numpy as jnp
from jax import lax
from jax.experimental import pallas as pl
from jax.experimental.pallas import tpu as pltpu
from jax.experimental.pallas import tpu_sc as plsc

N = 50000
D = 128
FE = 9
NF = 128
G = 32
EPS = 1e-5

NW = 32
NP = 50176
PW = NP // NW
C = 224
NCHUNK = PW // C

STATS_BN = 2000
MM_BN = 512


def _stats_body(x_ref, sum_ref, sq_ref):
    i = pl.program_id(0)
    x = x_ref[...]
    s = jnp.sum(x, axis=0, keepdims=True)
    q = jnp.sum(x * x, axis=0, keepdims=True)

    @pl.when(i == 0)
    def _():
        sum_ref[...] = s
        sq_ref[...] = q

    @pl.when(i != 0)
    def _():
        sum_ref[...] += s
        sq_ref[...] += q


def _mm_body(x_ref, scale_ref, shift_ref, w_ref, bb_ref, y_ref):
    x = x_ref[...]
    xn = jnp.maximum(x * scale_ref[...] + shift_ref[...], 0.0)
    xb = xn.astype(jnp.bfloat16)
    y = lax.dot_general(xb, w_ref[...], (((1,), (0,)), ((), ())),
                        preferred_element_type=jnp.float32)
    y_ref[...] = y + bb_ref[...]


def _sc_body(y_hbm, idx_hbm, out_hbm, idx_v, acc_v, rows_v, sem):
    wid = lax.axis_index("s") * 2 + lax.axis_index("c")
    base = wid * PW

    @pl.loop(0, NCHUNK)
    def _(ci):
        off = base + ci * C
        pltpu.sync_copy(idx_hbm.at[pl.ds(off, C)], idx_v)
        pltpu.async_copy(y_hbm.at[idx_v], acc_v, sem).wait()
        for k in range(1, FE):
            pltpu.sync_copy(idx_hbm.at[pl.ds(k * NP + off, C)], idx_v)
            pltpu.async_copy(y_hbm.at[idx_v], rows_v, sem).wait()

            @pl.loop(0, C)
            def _(r):
                for j in range(NF // 16):
                    plsc.addupdate(acc_v.at[r, pl.ds(j * 16, 16)],
                                   rows_v[r, pl.ds(j * 16, 16)])

        pltpu.sync_copy(acc_v, out_hbm.at[pl.ds(off, C)])


def kernel(lv, neighbor_idx, gamma, beta, W, b):
    f32 = jnp.float32
    sums, sqs = pl.pallas_call(
        _stats_body,
        grid=(N // STATS_BN,),
        in_specs=[pl.BlockSpec((STATS_BN, D), lambda i: (i, 0))],
        out_specs=[pl.BlockSpec((1, D), lambda i: (0, 0))] * 2,
        out_shape=[jax.ShapeDtypeStruct((1, D), f32)] * 2,
    )(lv)

    cs = sums.reshape(G, D // G)
    cq = sqs.reshape(G, D // G)
    cnt = f32(N * (D // G))
    mean = cs.sum(1) / cnt
    var = cq.sum(1) / cnt - mean * mean
    rstd = lax.rsqrt(var + EPS)
    g2 = gamma.reshape(G, D // G)
    b2 = beta.reshape(G, D // G)
    scale = (g2 * rstd[:, None]).reshape(1, D)
    shift = (b2 - g2 * (mean * rstd)[:, None]).reshape(1, D)

    w_all = W.reshape(FE, D, NF).transpose(1, 0, 2).reshape(D, FE * NF)
    w_all = w_all.astype(jnp.bfloat16)
    bias_rep = jnp.tile(b.reshape(1, NF) / f32(FE), (1, FE)).reshape(1, FE * NF)

    y_flat = pl.pallas_call(
        _mm_body,
        grid=(NP // MM_BN,),
        in_specs=[
            pl.BlockSpec((MM_BN, D), lambda i: (i, 0)),
            pl.BlockSpec((1, D), lambda i: (0, 0)),
            pl.BlockSpec((1, D), lambda i: (0, 0)),
            pl.BlockSpec((D, FE * NF), lambda i: (0, 0)),
            pl.BlockSpec((1, FE * NF), lambda i: (0, 0)),
        ],
        out_specs=pl.BlockSpec((MM_BN, FE * NF), lambda i: (i, 0)),
        out_shape=jax.ShapeDtypeStruct((NP, FE * NF), f32),
    )(lv, scale, shift, w_all, bias_rep)
    y_table = y_flat.reshape(NP * FE, NF)

    idx = neighbor_idx.astype(jnp.int32)
    idx2 = idx * FE + jnp.arange(FE, dtype=jnp.int32)[None, :]
    idx2 = jnp.pad(idx2, ((0, NP - N), (0, 0))).T.reshape(-1)

    mesh = plsc.VectorSubcoreMesh(core_axis_name="c", subcore_axis_name="s")
    sc_gather = pl.kernel(
        _sc_body,
        out_type=jax.ShapeDtypeStruct((NP, NF), f32),
        mesh=mesh,
        scratch_types=[
            pltpu.VMEM((C,), jnp.int32),
            pltpu.VMEM((C, NF), f32),
            pltpu.VMEM((C, NF), f32),
            pltpu.SemaphoreType.DMA,
        ],
    )
    out = sc_gather(y_table, idx2)
    return out[:N]


# speedup vs baseline: 3.7001x; 1.7054x over previous
import functools

import jax
import jax.numpy as jnp
from jax import lax
from jax.experimental import pallas as pl
from jax.experimental.pallas import tpu as pltpu
from jax.experimental.pallas import tpu_sc as plsc

N = 50000
D = 128
FE = 9
NF = 128
G = 32
EPS = 1e-5

NW = 32
NP = 50176
PW = NP // NW
C = 112
NCHUNK = PW // C
NIT = NCHUNK * FE

STATS_BN = 2000
MM_BN = 512


def _stats_body(x_ref, sum_ref, sq_ref):
    i = pl.program_id(0)
    x = x_ref[...]
    s = jnp.sum(x, axis=0, keepdims=True)
    q = jnp.sum(x * x, axis=0, keepdims=True)

    @pl.when(i == 0)
    def _():
        sum_ref[...] = s
        sq_ref[...] = q

    @pl.when(i != 0)
    def _():
        sum_ref[...] += s
        sq_ref[...] += q


def _mm_body(r3_ref, scale_ref, shift_ref, w_ref, b_ref, o_ref):
    o_ref[...] = jnp.zeros((MM_BN, NF), jnp.float32) + b_ref[...]
    for k in range(FE):
        x = r3_ref[k]
        xn = jnp.maximum(x * scale_ref[...] + shift_ref[...], 0.0)
        xb = xn.astype(jnp.bfloat16)
        o_ref[...] += lax.dot_general(xb, w_ref[k], (((1,), (0,)), ((), ())),
                                      preferred_element_type=jnp.float32)


def _sc_body(tbl_hbm, idx_hbm, rows_hbm, idx_all, buf0, buf1,
             sg0, sg1, sw0, sw1):
    wid = lax.axis_index("s") * 2 + lax.axis_index("c")
    base = wid * PW
    pltpu.sync_copy(idx_hbm.at[pl.ds(wid * (FE * PW), FE * PW)], idx_all)

    pairs = ((buf0, sg0, sw0), (buf1, sg1, sw1))

    def slots(it):
        k = it % FE
        ci = it // FE
        return k * PW + ci * C, k * NP + base + ci * C

    @pl.loop(0, NIT, step=2)
    def _(it0):
        for p in range(2):
            bufp, sgp, swp = pairs[p]
            it = it0 + p

            @pl.when(it >= 2)
            def _():
                pltpu.make_async_copy(bufp, rows_hbm.at[pl.ds(0, C)], swp).wait()

            voff, _ = slots(it)
            pltpu.async_copy(tbl_hbm.at[idx_all.at[pl.ds(voff, C)]], bufp, sgp)
        for p in range(2):
            bufp, sgp, swp = pairs[p]
            it = it0 + p
            voff, hoff = slots(it)
            pltpu.make_async_copy(
                tbl_hbm.at[idx_all.at[pl.ds(voff, C)]], bufp, sgp).wait()
            pltpu.async_copy(bufp, rows_hbm.at[pl.ds(hoff, C)], swp)

    for p in range(2):
        bufp, sgp, swp = pairs[p]
        pltpu.make_async_copy(bufp, rows_hbm.at[pl.ds(0, C)], swp).wait()


def kernel(lv, neighbor_idx, gamma, beta, W, b):
    f32 = jnp.float32

    idx = neighbor_idx.astype(jnp.int32)
    idx_wm = jnp.pad(idx, ((0, NP - N), (0, 0))).T
    idx_wm = idx_wm.reshape(FE, NW, PW).transpose(1, 0, 2).reshape(-1)

    mesh = plsc.VectorSubcoreMesh(core_axis_name="c", subcore_axis_name="s")
    sc_gather = pl.kernel(
        _sc_body,
        out_type=jax.ShapeDtypeStruct((FE * NP, D), f32),
        mesh=mesh,
        scratch_types=[
            pltpu.VMEM((FE * PW,), jnp.int32),
            pltpu.VMEM((C, D), f32),
            pltpu.VMEM((C, D), f32),
            pltpu.SemaphoreType.DMA,
            pltpu.SemaphoreType.DMA,
            pltpu.SemaphoreType.DMA,
            pltpu.SemaphoreType.DMA,
        ],
    )
    rows3 = sc_gather(lv, idx_wm).reshape(FE, NP, D)

    sums, sqs = pl.pallas_call(
        _stats_body,
        grid=(N // STATS_BN,),
        in_specs=[pl.BlockSpec((STATS_BN, D), lambda i: (i, 0))],
        out_specs=[pl.BlockSpec((1, D), lambda i: (0, 0))] * 2,
        out_shape=[jax.ShapeDtypeStruct((1, D), f32)] * 2,
    )(lv)

    cs = sums.reshape(G, D // G)
    cq = sqs.reshape(G, D // G)
    cnt = f32(N * (D // G))
    mean = cs.sum(1) / cnt
    var = cq.sum(1) / cnt - mean * mean
    rstd = lax.rsqrt(var + EPS)
    g2 = gamma.reshape(G, D // G)
    b2 = beta.reshape(G, D // G)
    scale = (g2 * rstd[:, None]).reshape(1, D)
    shift = (b2 - g2 * (mean * rstd)[:, None]).reshape(1, D)

    w3 = W.reshape(FE, D, NF).astype(jnp.bfloat16)
    out = pl.pallas_call(
        _mm_body,
        grid=(NP // MM_BN,),
        in_specs=[
            pl.BlockSpec((FE, MM_BN, D), lambda i: (0, i, 0)),
            pl.BlockSpec((1, D), lambda i: (0, 0)),
            pl.BlockSpec((1, D), lambda i: (0, 0)),
            pl.BlockSpec((FE, D, NF), lambda i: (0, 0, 0)),
            pl.BlockSpec((1, NF), lambda i: (0, 0)),
        ],
        out_specs=pl.BlockSpec((MM_BN, NF), lambda i: (i, 0)),
        out_shape=jax.ShapeDtypeStruct((NP, NF), f32),
    )(rows3, scale, shift, w3, b.reshape(1, NF))
    return out[:N]


# speedup vs baseline: 3.8353x; 1.0365x over previous
import functools

import jax
import jax.numpy as jnp
from jax import lax
from jax.experimental import pallas as pl
from jax.experimental.pallas import tpu as pltpu
from jax.experimental.pallas import tpu_sc as plsc

N = 50000
D = 128
FE = 9
NF = 128
G = 32
EPS = 1e-5

NW = 32
NP = 50176
PW = NP // NW
C = 112
NCHUNK = PW // C
NIT = NCHUNK * FE
NBUF = 6

STATS_BN = 2000
MM_BN = 512


def _stats_body(x_ref, sum_ref, sq_ref):
    i = pl.program_id(0)
    x = x_ref[...]
    s = jnp.sum(x, axis=0, keepdims=True)
    q = jnp.sum(x * x, axis=0, keepdims=True)

    @pl.when(i == 0)
    def _():
        sum_ref[...] = s
        sq_ref[...] = q

    @pl.when(i != 0)
    def _():
        sum_ref[...] += s
        sq_ref[...] += q


def _mm_body(r3_ref, scale_ref, shift_ref, w_ref, b_ref, o_ref):
    o_ref[...] = jnp.zeros((MM_BN, NF), jnp.float32) + b_ref[...]
    for k in range(FE):
        x = r3_ref[k]
        xn = jnp.maximum(x * scale_ref[...] + shift_ref[...], 0.0)
        xb = xn.astype(jnp.bfloat16)
        o_ref[...] += lax.dot_general(xb, w_ref[k], (((1,), (0,)), ((), ())),
                                      preferred_element_type=jnp.float32)


def _sc_body(tbl_hbm, idx_hbm, rows_hbm, idx_all, bufs, sgs, sws):
    wid = lax.axis_index("s") * 2 + lax.axis_index("c")
    base = wid * PW
    pltpu.sync_copy(idx_hbm.at[pl.ds(wid * (FE * PW), FE * PW)], idx_all)

    def slots(it):
        k = it % FE
        ci = it // FE
        return k * PW + ci * C, k * NP + base + ci * C

    @pl.loop(0, NIT, step=NBUF)
    def _(it0):
        for p in range(NBUF):
            it = it0 + p

            @pl.when(it >= NBUF)
            def _():
                pltpu.make_async_copy(
                    bufs[p], rows_hbm.at[pl.ds(0, C)], sws[p]).wait()

            voff, _ = slots(it)
            pltpu.async_copy(
                tbl_hbm.at[idx_all.at[pl.ds(voff, C)]], bufs[p], sgs[p])
        for p in range(NBUF):
            it = it0 + p
            voff, hoff = slots(it)
            pltpu.make_async_copy(
                tbl_hbm.at[idx_all.at[pl.ds(voff, C)]], bufs[p], sgs[p]).wait()
            pltpu.async_copy(bufs[p], rows_hbm.at[pl.ds(hoff, C)], sws[p])

    for p in range(NBUF):
        pltpu.make_async_copy(bufs[p], rows_hbm.at[pl.ds(0, C)], sws[p]).wait()


def kernel(lv, neighbor_idx, gamma, beta, W, b):
    f32 = jnp.float32

    idx = neighbor_idx.astype(jnp.int32)
    idx_wm = jnp.pad(idx, ((0, NP - N), (0, 0))).T
    idx_wm = idx_wm.reshape(FE, NW, PW).transpose(1, 0, 2).reshape(-1)

    mesh = plsc.VectorSubcoreMesh(core_axis_name="c", subcore_axis_name="s")
    sc_gather = pl.kernel(
        _sc_body,
        out_type=jax.ShapeDtypeStruct((FE * NP, D), f32),
        mesh=mesh,
        scratch_types=[
            pltpu.VMEM((FE * PW,), jnp.int32),
            [pltpu.VMEM((C, D), f32)] * NBUF,
            [pltpu.SemaphoreType.DMA] * NBUF,
            [pltpu.SemaphoreType.DMA] * NBUF,
        ],
    )
    rows3 = sc_gather(lv, idx_wm).reshape(FE, NP, D)

    sums, sqs = pl.pallas_call(
        _stats_body,
        grid=(N // STATS_BN,),
        in_specs=[pl.BlockSpec((STATS_BN, D), lambda i: (i, 0))],
        out_specs=[pl.BlockSpec((1, D), lambda i: (0, 0))] * 2,
        out_shape=[jax.ShapeDtypeStruct((1, D), f32)] * 2,
    )(lv)

    cs = sums.reshape(G, D // G)
    cq = sqs.reshape(G, D // G)
    cnt = f32(N * (D // G))
    mean = cs.sum(1) / cnt
    var = cq.sum(1) / cnt - mean * mean
    rstd = lax.rsqrt(var + EPS)
    g2 = gamma.reshape(G, D // G)
    b2 = beta.reshape(G, D // G)
    scale = (g2 * rstd[:, None]).reshape(1, D)
    shift = (b2 - g2 * (mean * rstd)[:, None]).reshape(1, D)

    w3 = W.reshape(FE, D, NF).astype(jnp.bfloat16)
    out = pl.pallas_call(
        _mm_body,
        grid=(NP // MM_BN,),
        in_specs=[
            pl.BlockSpec((FE, MM_BN, D), lambda i: (0, i, 0)),
            pl.BlockSpec((1, D), lambda i: (0, 0)),
            pl.BlockSpec((1, D), lambda i: (0, 0)),
            pl.BlockSpec((FE, D, NF), lambda i: (0, 0, 0)),
            pl.BlockSpec((1, NF), lambda i: (0, 0)),
        ],
        out_specs=pl.BlockSpec((MM_BN, NF), lambda i: (i, 0)),
        out_shape=jax.ShapeDtypeStruct((NP, NF), f32),
    )(rows3, scale, shift, w3, b.reshape(1, NF))
    return out[:N]
